# baseline (speedup 1.0000x reference)
import functools

import jax
import jax.numpy as jnp
from jax import lax
from jax.experimental import pallas as pl
from jax.experimental.pallas import tpu as pltpu
from jax.experimental.pallas import tpu_sc as plsc

WINDOW = 64
STRIDE = 32
NSEM = 4


def _sliding_window_sc(inp_hbm, out_hbm, sems):
    nc = 2
    wid = lax.axis_index("s") * nc + lax.axis_index("c")
    osz = out_hbm.shape[0]
    nblocks = inp_hbm.shape[0] // STRIDE
    per = nblocks // 32

    lo = wid * per

    def descs(k):
        b = lo + k
        src = inp_hbm.at[pl.ds(b * STRIDE, STRIDE)]
        sem = sems[k % NSEM]
        w1 = pltpu.make_async_copy(src, out_hbm.at[b, pl.ds(0, STRIDE)], sem)
        w2 = pltpu.make_async_copy(
            src, out_hbm.at[b - 1, pl.ds(STRIDE, STRIDE)], sem
        )
        return b, w1, w2

    def fire(k):
        b, w1, w2 = descs(k)

        @pl.when(b < osz)
        def _():
            w1.start()

        @pl.when(b > 0)
        def _():
            w2.start()

    def drain(k):
        b, w1, w2 = descs(k)

        @pl.when(b < osz)
        def _():
            w1.wait()

        @pl.when(b > 0)
        def _():
            w2.wait()

    depth = 2 * NSEM
    for k in range(per):
        if k - depth >= 0:
            drain(k - depth)
        fire(k)
    for k in range(max(0, per - depth), per):
        drain(k)


def kernel(input):
    T = input.shape[0]
    osz = (T - WINDOW) // STRIDE + 1
    D = input.shape[1]
    run = functools.partial(
        pl.kernel,
        mesh=plsc.VectorSubcoreMesh(core_axis_name="c", subcore_axis_name="s"),
        out_type=jax.ShapeDtypeStruct((osz, WINDOW, D), jnp.float32),
        scratch_types=[[pltpu.SemaphoreType.DMA] * NSEM],
    )(_sliding_window_sc)
    return run(input)


# speedup vs baseline: 37.6521x; 37.6521x over previous
import functools

import jax
import jax.numpy as jnp
from jax import lax
from jax.experimental import pallas as pl
from jax.experimental.pallas import tpu as pltpu
from jax.experimental.pallas import tpu_sc as plsc

WINDOW = 64
STRIDE = 32
NB = 6
AHEAD = 3


def _sliding_window_sc(inp_hbm, out_hbm, buf, rsems, wsems):
    nc = 2
    wid = lax.axis_index("s") * nc + lax.axis_index("c")
    osz = out_hbm.shape[0]
    nblocks = inp_hbm.shape[0] // STRIDE
    per = nblocks // 32

    lo = wid * per

    def read(k):
        slot = k % NB
        return pltpu.async_copy(
            inp_hbm.at[pl.ds((lo + k) * STRIDE, STRIDE)],
            buf.at[pl.ds(slot * STRIDE, STRIDE)],
            rsems[slot],
        )

    def write_descs(k):
        b = lo + k
        slot = k % NB
        src = buf.at[pl.ds(slot * STRIDE, STRIDE)]
        w1 = pltpu.make_async_copy(src, out_hbm.at[b, pl.ds(0, STRIDE)], wsems[slot])
        w2 = pltpu.make_async_copy(
            src, out_hbm.at[b - 1, pl.ds(STRIDE, STRIDE)], wsems[slot]
        )
        return b, w1, w2

    def fire_writes(k):
        b, w1, w2 = write_descs(k)

        @pl.when(b < osz)
        def _():
            w1.start()

        @pl.when(b > 0)
        def _():
            w2.start()

    def drain_writes(k):
        b, w1, w2 = write_descs(k)

        @pl.when(b < osz)
        def _():
            w1.wait()

        @pl.when(b > 0)
        def _():
            w2.wait()

    reads = {}
    for k in range(min(AHEAD, per)):
        reads[k] = read(k)
    for k in range(per):
        nk = k + AHEAD
        if nk < per:
            if nk - NB >= 0:
                drain_writes(nk - NB)
            reads[nk] = read(nk)
        reads[k].wait()
        fire_writes(k)
    for k in range(max(0, per - NB), per):
        drain_writes(k)


def kernel(input):
    T = input.shape[0]
    osz = (T - WINDOW) // STRIDE + 1
    D = input.shape[1]
    run = functools.partial(
        pl.kernel,
        mesh=plsc.VectorSubcoreMesh(core_axis_name="c", subcore_axis_name="s"),
        out_type=jax.ShapeDtypeStruct((osz, WINDOW, D), jnp.float32),
        scratch_types=[
            pltpu.VMEM((NB * STRIDE, D), jnp.float32),
            [pltpu.SemaphoreType.DMA] * NB,
            [pltpu.SemaphoreType.DMA] * NB,
        ],
    )(_sliding_window_sc)
    return run(input)
